# Initial kernel scaffold; baseline (speedup 1.0000x reference)
import functools

import jax
import jax.numpy as jnp
from jax import lax
from jax.experimental import pallas as pl
from jax.experimental.pallas import tpu as pltpu
from jax.experimental.pallas import tpu_sc as plsc

N = 10000
E = 320000
F_IN = 128
H = 256
HALF = 128
NG = 64
NC = 10

L_CHUNKS = 158
L_TOTAL = 16 * L_CHUNKS * 128
D_CHUNKS = 79
D_TOTAL = 32 * D_CHUNKS * 128
ACC_ROWS = N + 8
HIST_ROWS = N + 16

_mesh = plsc.VectorSubcoreMesh(core_axis_name="c", subcore_axis_name="s")


@functools.partial(
    pl.kernel,
    mesh=_mesh,
    out_type=jax.ShapeDtypeStruct((2, HIST_ROWS, 16), jnp.float32),
    scratch_types=[
        pltpu.VMEM((D_CHUNKS, 128), jnp.int32),
        pltpu.VMEM((128, 16), jnp.float32),
        pltpu.VMEM_SHARED((HIST_ROWS, 16), jnp.float32),
    ],
)
def _sc_degree(src_hbm, ones_hbm, zeros_hbm, out_hbm, idx_v, ones_v, hist):
    c = lax.axis_index("c")
    s = lax.axis_index("s")
    wid = s * 2 + c
    pltpu.sync_copy(src_hbm.at[wid], idx_v)
    pltpu.sync_copy(ones_hbm, ones_v)

    @pl.when(s == 0)
    def _():
        pltpu.sync_copy(zeros_hbm, hist)

    plsc.subcore_barrier()

    def body(j, carry):
        pltpu.sync_copy(ones_v, hist.at[idx_v.at[j]], add=True)
        return carry

    lax.fori_loop(0, D_CHUNKS, body, 0)
    plsc.subcore_barrier()
    rows = HIST_ROWS // 16
    pltpu.sync_copy(hist.at[pl.ds(s * rows, rows)],
                    out_hbm.at[c, pl.ds(s * rows, rows)])


@functools.partial(
    pl.kernel,
    mesh=_mesh,
    out_type=jax.ShapeDtypeStruct((2, N, HALF), jnp.float32),
    scratch_types=[
        pltpu.VMEM((L_CHUNKS, 128), jnp.int32),
        pltpu.VMEM((L_CHUNKS, 128), jnp.int32),
        pltpu.VMEM((128, HALF), jnp.float32),
        pltpu.VMEM_SHARED((ACC_ROWS, HALF), jnp.float32),
        pltpu.SemaphoreType.DMA,
    ],
)
def _sc_gather(ya_hbm, yb_hbm, src_hbm, dst_hbm, zeros_hbm, out_hbm,
               sidx, didx, buf, acc, gsem):
    c = lax.axis_index("c")
    s = lax.axis_index("s")
    pltpu.sync_copy(src_hbm.at[s], sidx)
    pltpu.sync_copy(dst_hbm.at[s], didx)
    pltpu.sync_copy(zeros_hbm, buf)
    rows = N // 16
    for k in range(5):
        pltpu.sync_copy(buf.at[pl.ds(0, rows // 5)],
                        acc.at[pl.ds(s * rows + k * (rows // 5), rows // 5)])
    plsc.subcore_barrier()

    def run(tab):
        def body(j, carry):
            pltpu.async_copy(tab.at[sidx.at[j]], buf, gsem).wait()
            pltpu.sync_copy(buf, acc.at[didx.at[j]], add=True)
            return carry
        lax.fori_loop(0, L_CHUNKS, body, 0)

    @pl.when(c == 0)
    def _():
        run(ya_hbm)

    @pl.when(c == 1)
    def _():
        run(yb_hbm)

    plsc.subcore_barrier()
    pltpu.sync_copy(acc.at[pl.ds(s * rows, rows)],
                    out_hbm.at[c, pl.ds(s * rows, rows)])


def _bn(h):
    mu = jnp.mean(h, axis=0, keepdims=True)
    xc = h - mu
    var = jnp.mean(xc * xc, axis=0, keepdims=True)
    return xc * lax.rsqrt(var + 1e-5) + 1e-4


def _dinv(degp_ref):
    d0 = degp_ref[0, :N, 0:1]
    d1 = degp_ref[1, :N, 0:1]
    return lax.rsqrt(d0 + d1 + 1.0)


def _dot(a, b):
    return jnp.dot(a, b, preferred_element_type=jnp.float32)


def _pre_body(x_ref, w_ref, o_ref):
    o_ref[...] = jnp.maximum(_dot(_bn(x_ref[...]), w_ref[...]), 0.0)


def _l1_body(h_ref, degp_ref, w_ref, oa_ref, ob_ref):
    dinv = _dinv(degp_ref)
    yp = dinv * _dot(_bn(h_ref[...]), w_ref[...])
    oa_ref[...] = yp[:, :HALF]
    ob_ref[...] = yp[:, HALF:]


def _layer_body(s_ref, ya_ref, yb_ref, degp_ref, b_ref, w_ref, oa_ref, ob_ref):
    dinv = _dinv(degp_ref)
    b = b_ref[...]
    hA = jnp.maximum(dinv * (s_ref[0] + ya_ref[...]) + b[:, :HALF], 0.0)
    hB = jnp.maximum(dinv * (s_ref[1] + yb_ref[...]) + b[:, HALF:], 0.0)
    w = w_ref[...]
    y = _dot(_bn(hA), w[:HALF, :]) + _dot(_bn(hB), w[HALF:, :])
    yp = dinv * y
    oa_ref[...] = yp[:, :HALF]
    ob_ref[...] = yp[:, HALF:]


def _sigmoid(z):
    return 1.0 / (1.0 + jnp.exp(-z))


def _post_body(s_ref, ya_ref, yb_ref, degp_ref, b_ref, batch_ref,
               wihT_ref, whhT_ref, bsum_ref, wfc_ref, bfc_ref,
               wcls_ref, bcls_ref, o_ref, xw_ref, ys_ref):
    dinv = _dinv(degp_ref)
    b = b_ref[...]
    hA = jnp.maximum(dinv * (s_ref[0] + ya_ref[...]) + b[:, :HALF], 0.0)
    hB = jnp.maximum(dinv * (s_ref[1] + yb_ref[...]) + b[:, HALF:], 0.0)
    seg = (lax.broadcasted_iota(jnp.int32, (NG, N), 0)
           == batch_ref[...]).astype(jnp.float32)
    g = jnp.concatenate([_dot(seg, hA), _dot(seg, hB)], axis=1)
    xw_ref[...] = _dot(g, wihT_ref[...]) + bsum_ref[...]
    whhT = whhT_ref[...]

    def step(t, carry):
        h, cc = carry
        gv = xw_ref[pl.ds(t, 1), :] + _dot(h, whhT)
        i = _sigmoid(gv[:, 0:H])
        f = _sigmoid(gv[:, H:2 * H])
        gg = jnp.tanh(gv[:, 2 * H:3 * H])
        o = _sigmoid(gv[:, 3 * H:4 * H])
        cc = f * cc + i * gg
        h = o * jnp.tanh(cc)
        ys_ref[pl.ds(t, 1), :] = h
        return (h, cc)

    z = jnp.zeros((1, H), jnp.float32)
    lax.fori_loop(0, NG, step, (z, z))
    y2 = jnp.maximum(_dot(_bn(ys_ref[...]), wfc_ref[...]) + bfc_ref[...], 0.0)
    logits = _dot(_bn(y2), wcls_ref[...]) + bcls_ref[...]
    m = jnp.max(logits, axis=1, keepdims=True)
    lse = jnp.log(jnp.sum(jnp.exp(logits - m), axis=1, keepdims=True)) + m
    o_ref[...] = logits - lse


def _sds(*shape):
    return jax.ShapeDtypeStruct(shape, jnp.float32)


def _tc_pre(x, W_feat):
    return pl.pallas_call(_pre_body, out_shape=_sds(N, H))(x, W_feat)


def _tc_l1(h0, degp, W1):
    return pl.pallas_call(
        _l1_body, out_shape=[_sds(N, HALF), _sds(N, HALF)])(h0, degp, W1)


def _tc_layer(s, ya, yb, degp, b2d, W):
    return pl.pallas_call(
        _layer_body,
        out_shape=[_sds(N, HALF), _sds(N, HALF)])(s, ya, yb, degp, b2d, W)


def _tc_post(s, ya, yb, degp, b2d, batch2d, wihT, whhT, bsum, wfc, bfc,
             wcls, bcls):
    return pl.pallas_call(
        _post_body,
        out_shape=_sds(NG, NC),
        scratch_shapes=[pltpu.VMEM((NG, 4 * H), jnp.float32),
                        pltpu.VMEM((NG, H), jnp.float32)],
    )(s, ya, yb, degp, b2d, batch2d, wihT, whhT, bsum, wfc, bfc, wcls, bcls)


def kernel(x, edge_index, batch, W_feat, W1, b1, W2, b2, W3, b3,
           W_ih, W_hh, b_ih, b_hh, W_fc, b_fc, W_cls, b_cls):
    ei = edge_index.astype(jnp.int32)
    src, dst = ei[0], ei[1]

    pad_l = L_TOTAL - E
    src_l = jnp.concatenate(
        [src, jnp.zeros((pad_l,), jnp.int32)]).reshape(16, L_CHUNKS, 128)
    dst_l = jnp.concatenate(
        [dst, jnp.full((pad_l,), N, jnp.int32)]).reshape(16, L_CHUNKS, 128)
    src_d = jnp.concatenate(
        [src, jnp.full((D_TOTAL - E,), N, jnp.int32)]).reshape(32, D_CHUNKS, 128)

    ones16 = jnp.ones((128, 16), jnp.float32)
    zhist = jnp.zeros((HIST_ROWS, 16), jnp.float32)
    z128 = jnp.zeros((128, HALF), jnp.float32)
    batch2d = batch.astype(jnp.int32).reshape(1, N)

    degp = _sc_degree(src_d, ones16, zhist)
    h0 = _tc_pre(x, W_feat)

    ya, yb = _tc_l1(h0, degp, W1)
    s1 = _sc_gather(ya, yb, src_l, dst_l, z128)
    ya, yb = _tc_layer(s1, ya, yb, degp, b1.reshape(1, H), W2)
    s2 = _sc_gather(ya, yb, src_l, dst_l, z128)
    ya, yb = _tc_layer(s2, ya, yb, degp, b2.reshape(1, H), W3)
    s3 = _sc_gather(ya, yb, src_l, dst_l, z128)

    return _tc_post(
        s3, ya, yb, degp, b3.reshape(1, H), batch2d,
        W_ih.T, W_hh.T, (b_ih + b_hh).reshape(1, 4 * H),
        W_fc, b_fc.reshape(1, H), W_cls, b_cls.reshape(1, NC))


# speedup vs baseline: 4.4499x; 4.4499x over previous
import functools

import jax
import jax.numpy as jnp
from jax import lax
from jax.experimental import pallas as pl
from jax.experimental.pallas import tpu as pltpu
from jax.experimental.pallas import tpu_sc as plsc

N = 10000
E = 320000
F_IN = 128
H = 256
HALF = 128
NG = 64
NC = 10

L_CHUNKS = 158
L_TOTAL = 16 * L_CHUNKS * 128
SPLIT = 5120
PH_ROWS = 5248


def _degree_body(src0_hbm, src1_hbm, zeros_hbm, ones_hbm, out_hbm,
                 idx_v, buf, acc):
    c = lax.axis_index("c")
    s = lax.axis_index("s")
    rows = PH_ROWS // 16

    @pl.when(c == 0)
    def _():
        pltpu.sync_copy(src0_hbm.at[s], idx_v)

    @pl.when(c == 1)
    def _():
        pltpu.sync_copy(src1_hbm.at[s], idx_v)

    pltpu.sync_copy(zeros_hbm, buf)
    for k, n in enumerate((128, 128, 72)):
        pltpu.sync_copy(buf.at[pl.ds(0, n)],
                        acc.at[pl.ds(s * rows + k * 128, n)])
    pltpu.sync_copy(ones_hbm, buf)
    plsc.subcore_barrier()

    def body(j, carry):
        pltpu.sync_copy(buf, acc.at[idx_v.at[j]], add=True)
        return carry

    lax.fori_loop(0, L_CHUNKS, body, 0)
    plsc.subcore_barrier()
    pltpu.sync_copy(acc.at[pl.ds(s * rows, rows)],
                    out_hbm.at[c, pl.ds(s * rows, rows)])


def _gather_body(ya_hbm, yb_hbm, src_hbm, dst0_hbm, dst1_hbm, zeros_hbm,
                 out_hbm, sidx, didx, buf, acc, gsem):
    c = lax.axis_index("c")
    s = lax.axis_index("s")
    pltpu.sync_copy(src_hbm.at[s], sidx)
    rows = PH_ROWS // 16

    def run(tab):
        def body(j, carry):
            pltpu.async_copy(tab.at[sidx.at[j]], buf, gsem).wait()
            pltpu.sync_copy(buf, acc.at[didx.at[j]], add=True)
            return carry
        lax.fori_loop(0, L_CHUNKS, body, 0)

    for p, dst_hbm in enumerate((dst0_hbm, dst1_hbm)):
        pltpu.sync_copy(dst_hbm.at[s], didx)
        pltpu.sync_copy(zeros_hbm, buf)
        for k, n in enumerate((128, 128, 72)):
            pltpu.sync_copy(buf.at[pl.ds(0, n)],
                            acc.at[pl.ds(s * rows + k * 128, n)])
        plsc.subcore_barrier()

        @pl.when(c == 0)
        def _():
            run(ya_hbm)

        @pl.when(c == 1)
        def _():
            run(yb_hbm)

        plsc.subcore_barrier()
        pltpu.sync_copy(acc.at[pl.ds(s * rows, rows)],
                        out_hbm.at[c, p, pl.ds(s * rows, rows)])


@functools.cache
def _sc_kernels():
    mesh = plsc.VectorSubcoreMesh(core_axis_name="c", subcore_axis_name="s")
    deg = functools.partial(
        pl.kernel,
        mesh=mesh,
        out_type=jax.ShapeDtypeStruct((2, PH_ROWS, 128), jnp.float32),
        scratch_types=[
            pltpu.VMEM((L_CHUNKS, 128), jnp.int32),
            pltpu.VMEM((128, 128), jnp.float32),
            pltpu.VMEM_SHARED((PH_ROWS, 128), jnp.float32),
        ],
    )(_degree_body)
    gat = functools.partial(
        pl.kernel,
        mesh=mesh,
        out_type=jax.ShapeDtypeStruct((2, 2, PH_ROWS, HALF), jnp.float32),
        scratch_types=[
            pltpu.VMEM((L_CHUNKS, 128), jnp.int32),
            pltpu.VMEM((L_CHUNKS, 128), jnp.int32),
            pltpu.VMEM((128, HALF), jnp.float32),
            pltpu.VMEM_SHARED((PH_ROWS, HALF), jnp.float32),
            pltpu.SemaphoreType.DMA,
        ],
    )(_gather_body)
    return deg, gat


def _sc_degree(src_p0, src_p1, z128, ones128):
    return _sc_kernels()[0](src_p0, src_p1, z128, ones128)


def _sc_gather(ya, yb, src_l, dst_l0, dst_l1, z128):
    return _sc_kernels()[1](ya, yb, src_l, dst_l0, dst_l1, z128)


def _bn(h):
    mu = jnp.mean(h, axis=0, keepdims=True)
    xc = h - mu
    var = jnp.mean(xc * xc, axis=0, keepdims=True)
    return xc * lax.rsqrt(var + 1e-5) + 1e-4


def _dinv(deg_ref):
    return lax.rsqrt(deg_ref[...] + 1.0)


def _dot(a, b):
    return jnp.dot(a, b, preferred_element_type=jnp.float32)


def _scat(s_ref, ci):
    return jnp.concatenate(
        [s_ref[ci, 0, :SPLIT, :], s_ref[ci, 1, :N - SPLIT, :]], axis=0)


def _pre_body(x_ref, w_ref, o_ref):
    o_ref[...] = jnp.maximum(_dot(_bn(x_ref[...]), w_ref[...]), 0.0)


def _l1_body(h_ref, degp_ref, w_ref, oa_ref, ob_ref):
    dinv = _dinv(degp_ref)
    yp = dinv * _dot(_bn(h_ref[...]), w_ref[...])
    oa_ref[...] = yp[:, :HALF]
    ob_ref[...] = yp[:, HALF:]


def _layer_body(s_ref, ya_ref, yb_ref, degp_ref, b_ref, w_ref, oa_ref, ob_ref):
    dinv = _dinv(degp_ref)
    b = b_ref[...]
    hA = jnp.maximum(dinv * (_scat(s_ref, 0) + ya_ref[...]) + b[:, :HALF], 0.0)
    hB = jnp.maximum(dinv * (_scat(s_ref, 1) + yb_ref[...]) + b[:, HALF:], 0.0)
    w = w_ref[...]
    y = _dot(_bn(hA), w[:HALF, :]) + _dot(_bn(hB), w[HALF:, :])
    yp = dinv * y
    oa_ref[...] = yp[:, :HALF]
    ob_ref[...] = yp[:, HALF:]


def _sigmoid(z):
    return 1.0 / (1.0 + jnp.exp(-z))


def _post_body(s_ref, ya_ref, yb_ref, degp_ref, b_ref, batch_ref,
               wihT_ref, whhT_ref, bsum_ref, wfc_ref, bfc_ref,
               wcls_ref, bcls_ref, o_ref, xw_ref, ys_ref):
    dinv = _dinv(degp_ref)
    b = b_ref[...]
    hA = jnp.maximum(dinv * (_scat(s_ref, 0) + ya_ref[...]) + b[:, :HALF], 0.0)
    hB = jnp.maximum(dinv * (_scat(s_ref, 1) + yb_ref[...]) + b[:, HALF:], 0.0)
    seg = (lax.broadcasted_iota(jnp.int32, (NG, N), 0)
           == batch_ref[...]).astype(jnp.float32)
    g = jnp.concatenate([_dot(seg, hA), _dot(seg, hB)], axis=1)
    xw_ref[...] = _dot(g, wihT_ref[...]) + bsum_ref[...]
    whhT = whhT_ref[...]

    def step(t, carry):
        h, cc = carry
        gv = xw_ref[pl.ds(t, 1), :] + _dot(h, whhT)
        i = _sigmoid(gv[:, 0:H])
        f = _sigmoid(gv[:, H:2 * H])
        gg = jnp.tanh(gv[:, 2 * H:3 * H])
        o = _sigmoid(gv[:, 3 * H:4 * H])
        cc = f * cc + i * gg
        h = o * jnp.tanh(cc)
        ys_ref[pl.ds(t, 1), :] = h
        return (h, cc)

    z = jnp.zeros((1, H), jnp.float32)
    lax.fori_loop(0, NG, step, (z, z))
    y2 = jnp.maximum(_dot(_bn(ys_ref[...]), wfc_ref[...]) + bfc_ref[...], 0.0)
    logits = _dot(_bn(y2), wcls_ref[...]) + bcls_ref[...]
    m = jnp.max(logits, axis=1, keepdims=True)
    lse = jnp.log(jnp.sum(jnp.exp(logits - m), axis=1, keepdims=True)) + m
    o_ref[...] = logits - lse


def _sds(*shape):
    return jax.ShapeDtypeStruct(shape, jnp.float32)


def _tc_pre(x, W_feat):
    return pl.pallas_call(_pre_body, out_shape=_sds(N, H))(x, W_feat)


def _tc_l1(h0, degp, W1):
    return pl.pallas_call(
        _l1_body, out_shape=[_sds(N, HALF), _sds(N, HALF)])(h0, degp, W1)


def _tc_layer(s, ya, yb, degp, b2d, W):
    return pl.pallas_call(
        _layer_body,
        out_shape=[_sds(N, HALF), _sds(N, HALF)])(s, ya, yb, degp, b2d, W)


def _tc_post(s, ya, yb, degp, b2d, batch2d, wihT, whhT, bsum, wfc, bfc,
             wcls, bcls):
    return pl.pallas_call(
        _post_body,
        out_shape=_sds(NG, NC),
        scratch_shapes=[pltpu.VMEM((NG, 4 * H), jnp.float32),
                        pltpu.VMEM((NG, H), jnp.float32)],
    )(s, ya, yb, degp, b2d, batch2d, wihT, whhT, bsum, wfc, bfc, wcls, bcls)


def kernel(x, edge_index, batch, W_feat, W1, b1, W2, b2, W3, b3,
           W_ih, W_hh, b_ih, b_hh, W_fc, b_fc, W_cls, b_cls):
    ei = edge_index.astype(jnp.int32)
    src, dst = ei[0], ei[1]

    pad_l = L_TOTAL - E
    src_l = jnp.concatenate(
        [src, jnp.zeros((pad_l,), jnp.int32)]).reshape(16, L_CHUNKS, 128)
    dpad = jnp.full((pad_l,), SPLIT, jnp.int32)
    d0 = jnp.concatenate([jnp.where(dst < SPLIT, dst, SPLIT), dpad])
    d1 = jnp.concatenate([jnp.where(dst >= SPLIT, dst - SPLIT, SPLIT), dpad])
    dst_l0 = d0.reshape(16, L_CHUNKS, 128)
    dst_l1 = d1.reshape(16, L_CHUNKS, 128)
    s0 = jnp.concatenate([jnp.where(src < SPLIT, src, SPLIT), dpad])
    s1 = jnp.concatenate([jnp.where(src >= SPLIT, src - SPLIT, SPLIT), dpad])
    src_p0 = s0.reshape(16, L_CHUNKS, 128)
    src_p1 = s1.reshape(16, L_CHUNKS, 128)

    z128 = jnp.zeros((128, HALF), jnp.float32)
    ones128 = jnp.ones((128, 128), jnp.float32)
    batch2d = batch.astype(jnp.int32).reshape(1, N)

    degh = _sc_degree(src_p0, src_p1, z128, ones128)
    degp = jnp.concatenate(
        [degh[0, :SPLIT, :1], degh[1, :N - SPLIT, :1]], axis=0)
    h0 = _tc_pre(x, W_feat)

    ya, yb = _tc_l1(h0, degp, W1)
    s1 = _sc_gather(ya, yb, src_l, dst_l0, dst_l1, z128)
    ya, yb = _tc_layer(s1, ya, yb, degp, b1.reshape(1, H), W2)
    s2 = _sc_gather(ya, yb, src_l, dst_l0, dst_l1, z128)
    ya, yb = _tc_layer(s2, ya, yb, degp, b2.reshape(1, H), W3)
    s3 = _sc_gather(ya, yb, src_l, dst_l0, dst_l1, z128)

    return _tc_post(
        s3, ya, yb, degp, b3.reshape(1, H), batch2d,
        W_ih.T, W_hh.T, (b_ih + b_hh).reshape(1, 4 * H),
        W_fc, b_fc.reshape(1, H), W_cls, b_cls.reshape(1, NC))


# speedup vs baseline: 4.6426x; 1.0433x over previous
import functools

import jax
import jax.numpy as jnp
from jax import lax
from jax.experimental import pallas as pl
from jax.experimental.pallas import tpu as pltpu
from jax.experimental.pallas import tpu_sc as plsc

N = 10000
E = 320000
F_IN = 128
H = 256
HALF = 128
NG = 64
NC = 10

L_CHUNKS = 158
L_TOTAL = 16 * L_CHUNKS * 128
SPLIT = 5120
PH_ROWS = 5248


def _degree_body(src0_hbm, src1_hbm, zeros_hbm, ones_hbm, out_hbm,
                 idx_v, buf, acc):
    c = lax.axis_index("c")
    s = lax.axis_index("s")
    rows = PH_ROWS // 16

    @pl.when(c == 0)
    def _():
        pltpu.sync_copy(src0_hbm.at[s], idx_v)

    @pl.when(c == 1)
    def _():
        pltpu.sync_copy(src1_hbm.at[s], idx_v)

    pltpu.sync_copy(zeros_hbm, buf)
    for k, n in enumerate((128, 128, 72)):
        pltpu.sync_copy(buf.at[pl.ds(0, n)],
                        acc.at[pl.ds(s * rows + k * 128, n)])
    pltpu.sync_copy(ones_hbm, buf)
    plsc.subcore_barrier()

    def body(j, carry):
        pltpu.sync_copy(buf, acc.at[idx_v.at[j]], add=True)
        return carry

    lax.fori_loop(0, L_CHUNKS, body, 0)
    plsc.subcore_barrier()
    pltpu.sync_copy(acc.at[pl.ds(s * rows, rows)],
                    out_hbm.at[c, pl.ds(s * rows, rows)])


def _gather_body(ya_hbm, yb_hbm, src_hbm, dst0_hbm, dst1_hbm, zeros_hbm,
                 out_hbm, sidx, didx, buf, buf2, acc, gsem, gsem2, ssem,
                 ssem2):
    c = lax.axis_index("c")
    s = lax.axis_index("s")
    pltpu.sync_copy(src_hbm.at[s], sidx)
    rows = PH_ROWS // 16

    def run(tab):
        def body(i, carry):
            j = 2 * i
            g0 = pltpu.async_copy(tab.at[sidx.at[j]], buf, gsem)
            g1 = pltpu.async_copy(tab.at[sidx.at[j + 1]], buf2, gsem2)
            g0.wait()
            pltpu.sync_copy(buf, acc.at[didx.at[j]], add=True)
            g1.wait()
            pltpu.sync_copy(buf2, acc.at[didx.at[j + 1]], add=True)
            return carry
        lax.fori_loop(0, L_CHUNKS // 2, body, 0)

    for p, dst_hbm in enumerate((dst0_hbm, dst1_hbm)):
        pltpu.sync_copy(dst_hbm.at[s], didx)
        pltpu.sync_copy(zeros_hbm, buf)
        for k, n in enumerate((128, 128, 72)):
            pltpu.sync_copy(buf.at[pl.ds(0, n)],
                            acc.at[pl.ds(s * rows + k * 128, n)])
        plsc.subcore_barrier()

        @pl.when(c == 0)
        def _():
            run(ya_hbm)

        @pl.when(c == 1)
        def _():
            run(yb_hbm)

        plsc.subcore_barrier()
        pltpu.sync_copy(acc.at[pl.ds(s * rows, rows)],
                        out_hbm.at[c, p, pl.ds(s * rows, rows)])


@functools.cache
def _sc_kernels():
    mesh = plsc.VectorSubcoreMesh(core_axis_name="c", subcore_axis_name="s")
    deg = functools.partial(
        pl.kernel,
        mesh=mesh,
        out_type=jax.ShapeDtypeStruct((2, PH_ROWS, 128), jnp.float32),
        scratch_types=[
            pltpu.VMEM((L_CHUNKS, 128), jnp.int32),
            pltpu.VMEM((128, 128), jnp.float32),
            pltpu.VMEM_SHARED((PH_ROWS, 128), jnp.float32),
        ],
    )(_degree_body)
    gat = functools.partial(
        pl.kernel,
        mesh=mesh,
        out_type=jax.ShapeDtypeStruct((2, 2, PH_ROWS, HALF), jnp.float32),
        scratch_types=[
            pltpu.VMEM((L_CHUNKS, 128), jnp.int32),
            pltpu.VMEM((L_CHUNKS, 128), jnp.int32),
            pltpu.VMEM((128, HALF), jnp.float32),
            pltpu.VMEM((128, HALF), jnp.float32),
            pltpu.VMEM_SHARED((PH_ROWS, HALF), jnp.float32),
            pltpu.SemaphoreType.DMA,
            pltpu.SemaphoreType.DMA,
            pltpu.SemaphoreType.DMA,
            pltpu.SemaphoreType.DMA,
        ],
    )(_gather_body)
    return deg, gat


def _sc_degree(src_p0, src_p1, z128, ones128):
    return _sc_kernels()[0](src_p0, src_p1, z128, ones128)


def _sc_gather(ya, yb, src_l, dst_l0, dst_l1, z128):
    return _sc_kernels()[1](ya, yb, src_l, dst_l0, dst_l1, z128)


def _bn(h):
    mu = jnp.mean(h, axis=0, keepdims=True)
    xc = h - mu
    var = jnp.mean(xc * xc, axis=0, keepdims=True)
    return xc * lax.rsqrt(var + 1e-5) + 1e-4


def _dinv(deg_ref):
    return lax.rsqrt(deg_ref[...] + 1.0)


def _dot(a, b):
    return jnp.dot(a, b, preferred_element_type=jnp.float32)


def _scat(s_ref, ci):
    return jnp.concatenate(
        [s_ref[ci, 0, :SPLIT, :], s_ref[ci, 1, :N - SPLIT, :]], axis=0)


def _pre_body(x_ref, w_ref, o_ref):
    o_ref[...] = jnp.maximum(_dot(_bn(x_ref[...]), w_ref[...]), 0.0)


def _l1_body(h_ref, degp_ref, w_ref, oa_ref, ob_ref):
    dinv = _dinv(degp_ref)
    yp = dinv * _dot(_bn(h_ref[...]), w_ref[...])
    oa_ref[...] = yp[:, :HALF]
    ob_ref[...] = yp[:, HALF:]


def _layer_body(s_ref, ya_ref, yb_ref, degp_ref, b_ref, w_ref, oa_ref, ob_ref):
    dinv = _dinv(degp_ref)
    b = b_ref[...]
    hA = jnp.maximum(dinv * (_scat(s_ref, 0) + ya_ref[...]) + b[:, :HALF], 0.0)
    hB = jnp.maximum(dinv * (_scat(s_ref, 1) + yb_ref[...]) + b[:, HALF:], 0.0)
    w = w_ref[...]
    y = _dot(_bn(hA), w[:HALF, :]) + _dot(_bn(hB), w[HALF:, :])
    yp = dinv * y
    oa_ref[...] = yp[:, :HALF]
    ob_ref[...] = yp[:, HALF:]


def _sigmoid(z):
    return 1.0 / (1.0 + jnp.exp(-z))


def _post_body(s_ref, ya_ref, yb_ref, degp_ref, b_ref, batch_ref,
               wihT_ref, whhT_ref, bsum_ref, wfc_ref, bfc_ref,
               wcls_ref, bcls_ref, o_ref, xw_ref, ys_ref):
    dinv = _dinv(degp_ref)
    b = b_ref[...]
    hA = jnp.maximum(dinv * (_scat(s_ref, 0) + ya_ref[...]) + b[:, :HALF], 0.0)
    hB = jnp.maximum(dinv * (_scat(s_ref, 1) + yb_ref[...]) + b[:, HALF:], 0.0)
    seg = (lax.broadcasted_iota(jnp.int32, (NG, N), 0)
           == batch_ref[...]).astype(jnp.float32)
    g = jnp.concatenate([_dot(seg, hA), _dot(seg, hB)], axis=1)
    xw_ref[...] = _dot(g, wihT_ref[...]) + bsum_ref[...]
    whhT = whhT_ref[...]

    def step(t, carry):
        h, cc = carry
        gv = xw_ref[pl.ds(t, 1), :] + _dot(h, whhT)
        i = _sigmoid(gv[:, 0:H])
        f = _sigmoid(gv[:, H:2 * H])
        gg = jnp.tanh(gv[:, 2 * H:3 * H])
        o = _sigmoid(gv[:, 3 * H:4 * H])
        cc = f * cc + i * gg
        h = o * jnp.tanh(cc)
        ys_ref[pl.ds(t, 1), :] = h
        return (h, cc)

    z = jnp.zeros((1, H), jnp.float32)
    lax.fori_loop(0, NG, step, (z, z))
    y2 = jnp.maximum(_dot(_bn(ys_ref[...]), wfc_ref[...]) + bfc_ref[...], 0.0)
    logits = _dot(_bn(y2), wcls_ref[...]) + bcls_ref[...]
    m = jnp.max(logits, axis=1, keepdims=True)
    lse = jnp.log(jnp.sum(jnp.exp(logits - m), axis=1, keepdims=True)) + m
    o_ref[...] = logits - lse


def _sds(*shape):
    return jax.ShapeDtypeStruct(shape, jnp.float32)


def _tc_pre(x, W_feat):
    return pl.pallas_call(_pre_body, out_shape=_sds(N, H))(x, W_feat)


def _tc_l1(h0, degp, W1):
    return pl.pallas_call(
        _l1_body, out_shape=[_sds(N, HALF), _sds(N, HALF)])(h0, degp, W1)


def _tc_layer(s, ya, yb, degp, b2d, W):
    return pl.pallas_call(
        _layer_body,
        out_shape=[_sds(N, HALF), _sds(N, HALF)])(s, ya, yb, degp, b2d, W)


def _tc_post(s, ya, yb, degp, b2d, batch2d, wihT, whhT, bsum, wfc, bfc,
             wcls, bcls):
    return pl.pallas_call(
        _post_body,
        out_shape=_sds(NG, NC),
        scratch_shapes=[pltpu.VMEM((NG, 4 * H), jnp.float32),
                        pltpu.VMEM((NG, H), jnp.float32)],
    )(s, ya, yb, degp, b2d, batch2d, wihT, whhT, bsum, wfc, bfc, wcls, bcls)


def kernel(x, edge_index, batch, W_feat, W1, b1, W2, b2, W3, b3,
           W_ih, W_hh, b_ih, b_hh, W_fc, b_fc, W_cls, b_cls):
    ei = edge_index.astype(jnp.int32)
    src, dst = ei[0], ei[1]

    pad_l = L_TOTAL - E
    src_l = jnp.concatenate(
        [src, jnp.zeros((pad_l,), jnp.int32)]).reshape(16, L_CHUNKS, 128)
    dpad = jnp.full((pad_l,), SPLIT, jnp.int32)
    d0 = jnp.concatenate([jnp.where(dst < SPLIT, dst, SPLIT), dpad])
    d1 = jnp.concatenate([jnp.where(dst >= SPLIT, dst - SPLIT, SPLIT), dpad])
    dst_l0 = d0.reshape(16, L_CHUNKS, 128)
    dst_l1 = d1.reshape(16, L_CHUNKS, 128)
    s0 = jnp.concatenate([jnp.where(src < SPLIT, src, SPLIT), dpad])
    s1 = jnp.concatenate([jnp.where(src >= SPLIT, src - SPLIT, SPLIT), dpad])
    src_p0 = s0.reshape(16, L_CHUNKS, 128)
    src_p1 = s1.reshape(16, L_CHUNKS, 128)

    z128 = jnp.zeros((128, HALF), jnp.float32)
    ones128 = jnp.ones((128, 128), jnp.float32)
    batch2d = batch.astype(jnp.int32).reshape(1, N)

    degh = _sc_degree(src_p0, src_p1, z128, ones128)
    degp = jnp.concatenate(
        [degh[0, :SPLIT, :1], degh[1, :N - SPLIT, :1]], axis=0)
    h0 = _tc_pre(x, W_feat)

    ya, yb = _tc_l1(h0, degp, W1)
    s1 = _sc_gather(ya, yb, src_l, dst_l0, dst_l1, z128)
    ya, yb = _tc_layer(s1, ya, yb, degp, b1.reshape(1, H), W2)
    s2 = _sc_gather(ya, yb, src_l, dst_l0, dst_l1, z128)
    ya, yb = _tc_layer(s2, ya, yb, degp, b2.reshape(1, H), W3)
    s3 = _sc_gather(ya, yb, src_l, dst_l0, dst_l1, z128)

    return _tc_post(
        s3, ya, yb, degp, b3.reshape(1, H), batch2d,
        W_ih.T, W_hh.T, (b_ih + b_hh).reshape(1, 4 * H),
        W_fc, b_fc.reshape(1, H), W_cls, b_cls.reshape(1, NC))
